# Initial kernel scaffold; baseline (speedup 1.0000x reference)
import jax
import jax.numpy as jnp
from jax.experimental import pallas as pl


def kernel(x, edge_index, W0, a_src0, a_dst0, b0, W1, a_src1, a_dst1, b1, W2, a_src2, a_dst2, b2):
    raise NotImplementedError("write your pallas kernel here")


# speedup vs baseline: 330.4134x; 330.4134x over previous
import functools

import jax
import jax.numpy as jnp
from jax.experimental import pallas as pl
from jax.experimental.pallas import tpu as pltpu

B = 512
DIM = 128
G = 8
NB = B // G


def _dot(a, b, dims):
    return jax.lax.dot_general(a, b, (dims, ((), ())),
                               preferred_element_type=jnp.float32)


def _prep_kernel(src_ref, dst_ref, c1_ref):
    for g in range(G):
        s_row = jnp.bitwise_and(src_ref[g:g + 1, :], 63)
        d_row = jnp.bitwise_and(dst_ref[g:g + 1, :], 63)
        io = jax.lax.broadcasted_iota(jnp.int32, (64, 1024), 0)
        s_oh = (io == s_row).astype(jnp.bfloat16)
        d_oh = (io == d_row).astype(jnp.bfloat16)
        c = _dot(d_oh, s_oh, (((1,), (1,))))
        c1_ref[g * 64:(g + 1) * 64, :] = c


def _layer_kernel(h_ref, c_ref, stats_ref, w_ref, asrc_ref, adst_ref, b_ref,
                  hout_ref, stats_out_ref, mout_ref, *, n, has_bn, prev_rows):
    R = G * n
    h = h_ref[...]
    if has_bn:
        s0 = stats_ref[0:1, :]
        s1 = stats_ref[1:2, :]
        mean = s0 / prev_rows
        var = s1 / prev_rows - mean * mean
        h = (h - mean) * jax.lax.rsqrt(var + 1e-5)
    xl = _dot(h, w_ref[...], ((1,), (1,)))
    ad = _dot(xl, adst_ref[...], ((1,), (1,)))
    as_row = _dot(asrc_ref[...], xl, ((1,), (1,)))
    e = ad + as_row
    e = jnp.where(e > 0.0, e, 0.2 * e)

    cr = c_ref[...]
    tiled = jnp.concatenate([cr] * G, axis=1)
    row = jax.lax.broadcasted_iota(jnp.int32, (R, R), 0)
    col = jax.lax.broadcasted_iota(jnp.int32, (R, R), 1)
    same_graph = (row // n) == (col // n)
    cfull = jnp.where(same_graph, tiled, 0.0)
    cfull = cfull + (row == col).astype(jnp.float32)

    mask = cfull > 0.0
    em = jnp.where(mask, e, -1e30)
    m = jnp.max(em, axis=1, keepdims=True)
    ex = jnp.exp(em - m) * cfull
    den = jnp.sum(ex, axis=1, keepdims=True)
    num = _dot(ex, xl, ((1,), (0,)))
    out = jnp.maximum(num / den + b_ref[...], 0.0)

    hp = jnp.max(out.reshape(R // 2, 2, DIM), axis=1)
    hout_ref[...] = hp

    upd = jnp.concatenate(
        [jnp.sum(hp, axis=0, keepdims=True),
         jnp.sum(hp * hp, axis=0, keepdims=True)], axis=0)

    @pl.when(pl.program_id(0) == 0)
    def _():
        stats_out_ref[...] = upd

    @pl.when(pl.program_id(0) != 0)
    def _():
        stats_out_ref[...] = stats_out_ref[...] + upd

    if mout_ref is not None:
        rp = jnp.max(cr.reshape(R // 2, 2, n), axis=1)
        jo = jax.lax.broadcasted_iota(jnp.int32, (n, n // 2), 0)
        ko = jax.lax.broadcasted_iota(jnp.int32, (n, n // 2), 1)
        pmat = ((jo // 2) == ko).astype(jnp.float32)
        cp = _dot(rp, pmat, ((1,), (0,)))
        r2 = jax.lax.broadcasted_iota(jnp.int32, (R // 2, n // 2), 0)
        c2 = jax.lax.broadcasted_iota(jnp.int32, (R // 2, n // 2), 1)
        offdiag = (r2 % (n // 2)) != c2
        mout_ref[...] = jnp.where((cp > 0.0) & offdiag, 1.0, 0.0)


def _final_kernel(h_ref, stats_ref, out_ref, *, rows):
    s0 = stats_ref[0:1, :]
    s1 = stats_ref[1:2, :]
    mean = s0 / rows
    var = s1 / rows - mean * mean
    out_ref[...] = (h_ref[...] - mean) * jax.lax.rsqrt(var + 1e-5)


def _run_layer(h, c, stats, W, a_src, a_dst, b, *, n, has_bn, has_mnext):
    R = G * n
    prev_rows = B * n
    kern = functools.partial(
        _layer_kernel, n=n, has_bn=has_bn, prev_rows=float(prev_rows))
    if not has_mnext:
        kern2 = lambda *refs: kern(*refs, None)
    else:
        kern2 = kern
    const = lambda i: (0, 0)
    in_specs = [
        pl.BlockSpec((R, DIM), lambda i: (i, 0)),
        pl.BlockSpec((R, n), lambda i: (i, 0)),
        pl.BlockSpec((2, DIM), const),
        pl.BlockSpec((DIM, DIM), const),
        pl.BlockSpec((1, DIM), const),
        pl.BlockSpec((1, DIM), const),
        pl.BlockSpec((1, DIM), const),
    ]
    out_shapes = [
        jax.ShapeDtypeStruct((B * n // 2, DIM), jnp.float32),
        jax.ShapeDtypeStruct((2, DIM), jnp.float32),
    ]
    out_specs = [
        pl.BlockSpec((R // 2, DIM), lambda i: (i, 0)),
        pl.BlockSpec((2, DIM), const),
    ]
    if has_mnext:
        out_shapes.append(jax.ShapeDtypeStruct((B * n // 2, n // 2), jnp.float32))
        out_specs.append(pl.BlockSpec((R // 2, n // 2), lambda i: (i, 0)))
    res = pl.pallas_call(
        kern2,
        grid=(NB,),
        in_specs=in_specs,
        out_specs=out_specs,
        out_shape=out_shapes,
        compiler_params=pltpu.CompilerParams(
            dimension_semantics=("arbitrary",)),
    )(h, c, stats, W, a_src[None, :], a_dst[None, :], b[None, :])
    return res


def kernel(x, edge_index, W0, a_src0, a_dst0, b0, W1, a_src1, a_dst1, b1,
           W2, a_src2, a_dst2, b2):
    src = edge_index[0].reshape(B, 1024)
    dst = edge_index[1].reshape(B, 1024)

    c1 = pl.pallas_call(
        _prep_kernel,
        grid=(NB,),
        in_specs=[pl.BlockSpec((G, 1024), lambda i: (i, 0)),
                  pl.BlockSpec((G, 1024), lambda i: (i, 0))],
        out_specs=pl.BlockSpec((G * 64, 64), lambda i: (i, 0)),
        out_shape=jax.ShapeDtypeStruct((B * 64, 64), jnp.float32),
        compiler_params=pltpu.CompilerParams(
            dimension_semantics=("arbitrary",)),
    )(src, dst)

    stats0 = jnp.zeros((2, DIM), jnp.float32)
    h1, stats1, m2 = _run_layer(x, c1, stats0, W0, a_src0, a_dst0, b0,
                                n=64, has_bn=False, has_mnext=True)
    h2, stats2, m3 = _run_layer(h1, m2, stats1, W1, a_src1, a_dst1, b1,
                                n=32, has_bn=True, has_mnext=True)
    h3, stats3 = _run_layer(h2, m3, stats2, W2, a_src2, a_dst2, b2,
                            n=16, has_bn=True, has_mnext=False)

    out = pl.pallas_call(
        functools.partial(_final_kernel, rows=float(B * 8)),
        in_specs=[pl.BlockSpec((B * 8, DIM), lambda: (0, 0)),
                  pl.BlockSpec((2, DIM), lambda: (0, 0))],
        out_specs=pl.BlockSpec((B * 8, DIM), lambda: (0, 0)),
        out_shape=jax.ShapeDtypeStruct((B * 8, DIM), jnp.float32),
    )(h3, stats3)
    return out.reshape(B, 8 * DIM)


# speedup vs baseline: 397.5114x; 1.2031x over previous
import functools

import jax
import jax.numpy as jnp
from jax.experimental import pallas as pl
from jax.experimental.pallas import tpu as pltpu

B = 512
DIM = 128
G = 8
NB = B // G


def _dot(a, b, dims):
    return jax.lax.dot_general(a, b, (dims, ((), ())),
                               preferred_element_type=jnp.float32)


def _iota2(shape, d):
    return jax.lax.broadcasted_iota(jnp.int32, shape, d)


def _gat_block(h, cs, W, a_src, a_dst, b, n):
    R = G * n
    xl = _dot(h, W, ((1,), (1,)))
    ad = _dot(xl, a_dst, ((1,), (1,)))
    as_row = _dot(a_src, xl, ((1,), (1,)))
    eye = (_iota2((n, n), 0) == _iota2((n, n), 1)).astype(jnp.float32)
    outs = []
    for g in range(G):
        sl = slice(g * n, (g + 1) * n)
        e = ad[sl, :] + as_row[:, sl]
        e = jnp.where(e > 0.0, e, 0.2 * e)
        cg = cs[g] + eye
        em = jnp.where(cg > 0.0, e, -1e30)
        m = jnp.max(em, axis=1, keepdims=True)
        ex = jnp.exp(em - m) * cg
        den = jnp.sum(ex, axis=1, keepdims=True)
        num = _dot(ex, xl[sl, :], ((1,), (0,)))
        outs.append(jnp.maximum(num / den + b, 0.0))
    out = jnp.concatenate(outs, axis=0)
    hp = jnp.max(out.reshape(R // 2, 2, DIM), axis=1)
    return hp


def _pool_masks(cs, n):
    n2 = n // 2
    pr = ((_iota2((n2, n), 1) // 2) == _iota2((n2, n), 0)).astype(jnp.float32)
    pc = ((_iota2((n, n2), 0) // 2) == _iota2((n, n2), 1)).astype(jnp.float32)
    offdiag = (_iota2((n2, n2), 0) != _iota2((n2, n2), 1))
    out = []
    for c in cs:
        cp = _dot(_dot(pr, c, ((1,), (0,))), pc, ((1,), (0,)))
        out.append(jnp.where((cp > 0.0) & offdiag, 1.0, 0.0))
    return out


def _accum_stats(stats_ref, row, hp, is_first):
    upd = jnp.concatenate(
        [jnp.sum(hp, axis=0, keepdims=True),
         jnp.sum(hp * hp, axis=0, keepdims=True)], axis=0)

    @pl.when(is_first)
    def _():
        stats_ref[row:row + 2, :] = upd

    @pl.when(jnp.logical_not(is_first))
    def _():
        stats_ref[row:row + 2, :] = stats_ref[row:row + 2, :] + upd


def _bn(h, stats_ref, row, rows):
    s0 = stats_ref[row:row + 1, :]
    s1 = stats_ref[row + 1:row + 2, :]
    mean = s0 / rows
    var = s1 / rows - mean * mean
    return (h - mean) * jax.lax.rsqrt(var + 1e-5)


def _fused_kernel(x_ref, src_ref, dst_ref,
                  w0_ref, as0_ref, ad0_ref, b0_ref,
                  w1_ref, as1_ref, ad1_ref, b1_ref,
                  w2_ref, as2_ref, ad2_ref, b2_ref,
                  out_ref,
                  h1_s, h2_s, h3_s, m2_s, m3_s, stats_s):
    p = pl.program_id(0)
    i = pl.program_id(1)
    is_first = i == 0

    @pl.when(p == 0)
    def _pass0():
        io = _iota2((64, 1024), 0)
        cs = []
        for g in range(G):
            s_oh = (io == jnp.bitwise_and(src_ref[g:g + 1, :], 63)
                    ).astype(jnp.bfloat16)
            d_oh = (io == jnp.bitwise_and(dst_ref[g:g + 1, :], 63)
                    ).astype(jnp.bfloat16)
            cs.append(_dot(d_oh, s_oh, ((1,), (1,))))
        hp = _gat_block(x_ref[...], cs, w0_ref[...], as0_ref[...],
                        ad0_ref[...], b0_ref[...], 64)
        h1_s[pl.ds(i * 256, 256), :] = hp
        _accum_stats(stats_s, 0, hp, is_first)
        ms = _pool_masks(cs, 64)
        for g in range(G):
            m2_s[pl.ds(i * 256 + g * 32, 32), :] = ms[g]

    @pl.when(p == 1)
    def _pass1():
        h = _bn(h1_s[pl.ds(i * 256, 256), :], stats_s, 0, float(B * 32))
        cs = [m2_s[pl.ds(i * 256 + g * 32, 32), :] for g in range(G)]
        hp = _gat_block(h, cs, w1_ref[...], as1_ref[...],
                        ad1_ref[...], b1_ref[...], 32)
        h2_s[pl.ds(i * 128, 128), :] = hp
        _accum_stats(stats_s, 2, hp, is_first)
        ms = _pool_masks(cs, 32)
        for g in range(G):
            m3_s[pl.ds(i * 128 + g * 16, 16), :] = ms[g]

    @pl.when(p == 2)
    def _pass2():
        h = _bn(h2_s[pl.ds(i * 128, 128), :], stats_s, 2, float(B * 16))
        cs = [m3_s[pl.ds(i * 128 + g * 16, 16), :] for g in range(G)]
        hp = _gat_block(h, cs, w2_ref[...], as2_ref[...],
                        ad2_ref[...], b2_ref[...], 16)
        h3_s[pl.ds(i * 64, 64), :] = hp
        _accum_stats(stats_s, 4, hp, is_first)

    @pl.when(p == 3)
    def _pass3():
        out_ref[...] = _bn(h3_s[pl.ds(i * 64, 64), :], stats_s, 4,
                           float(B * 8))


def kernel(x, edge_index, W0, a_src0, a_dst0, b0, W1, a_src1, a_dst1, b1,
           W2, a_src2, a_dst2, b2):
    src = edge_index[0].reshape(B, 1024)
    dst = edge_index[1].reshape(B, 1024)

    first = lambda p, i: (jnp.where(p == 0, i, 0), 0)
    const = lambda p, i: (0, 0)
    last = lambda p, i: (jnp.where(p == 3, i, 0), 0)

    out = pl.pallas_call(
        _fused_kernel,
        grid=(4, NB),
        in_specs=[
            pl.BlockSpec((G * 64, DIM), first),
            pl.BlockSpec((G, 1024), first),
            pl.BlockSpec((G, 1024), first),
            pl.BlockSpec((DIM, DIM), const),
            pl.BlockSpec((1, DIM), const),
            pl.BlockSpec((1, DIM), const),
            pl.BlockSpec((1, DIM), const),
            pl.BlockSpec((DIM, DIM), const),
            pl.BlockSpec((1, DIM), const),
            pl.BlockSpec((1, DIM), const),
            pl.BlockSpec((1, DIM), const),
            pl.BlockSpec((DIM, DIM), const),
            pl.BlockSpec((1, DIM), const),
            pl.BlockSpec((1, DIM), const),
            pl.BlockSpec((1, DIM), const),
        ],
        out_specs=pl.BlockSpec((G * 8, DIM), last),
        out_shape=jax.ShapeDtypeStruct((B * 8, DIM), jnp.float32),
        scratch_shapes=[
            pltpu.VMEM((B * 32, DIM), jnp.float32),
            pltpu.VMEM((B * 16, DIM), jnp.float32),
            pltpu.VMEM((B * 8, DIM), jnp.float32),
            pltpu.VMEM((B * 32, 32), jnp.float32),
            pltpu.VMEM((B * 16, 16), jnp.float32),
            pltpu.VMEM((8, DIM), jnp.float32),
        ],
        compiler_params=pltpu.CompilerParams(
            dimension_semantics=("arbitrary", "arbitrary")),
    )(x, src, dst,
      W0, a_src0[None, :], a_dst0[None, :], b0[None, :],
      W1, a_src1[None, :], a_dst1[None, :], b1[None, :],
      W2, a_src2[None, :], a_dst2[None, :], b2[None, :])
    return out.reshape(B, 8 * DIM)
